# baseline (speedup 1.0000x reference)
import functools

import jax
import jax.numpy as jnp
from jax import lax
from jax.experimental import pallas as pl
from jax.experimental.pallas import tpu as pltpu
from jax.experimental.pallas import tpu_sc as plsc

B, T, D, L, V = 4096, 200, 64, 4, 100000
VE = V + 2
N = B * T
NC, NS = 2, 16
NW = NC * NS
PER_W = N // NW
CH = 128
NCH = PER_W // CH
GRP = 2
NROUND = NCH // GRP
NBUF = 2 * GRP

BC = 512
VEP = 100352
DP = 2 * D


def _aug_body(e0_ref, e1_ref, e2_ref, e3_ref, lv_ref, out_ref):
    lv = lv_ref[...]
    for l, e_ref in enumerate((e0_ref, e1_ref, e2_ref, e3_ref)):
        row = e_ref[...].T + lv[l, :]
        out_ref[l, :, :] = jnp.concatenate([row, row], axis=-1)


def _build_table(emb0, emb1, emb2, emb3, level_embed):
    grid = (VEP // BC,)
    aug = pl.pallas_call(
        _aug_body,
        grid=grid,
        in_specs=[pl.BlockSpec((D, BC), lambda i: (0, i)) for _ in range(4)]
        + [pl.BlockSpec((L, D), lambda i: (0, 0))],
        out_specs=pl.BlockSpec((L, BC, DP), lambda i: (0, i, 0)),
        out_shape=jax.ShapeDtypeStruct((L, VEP, DP), jnp.float32),
    )(emb0.T, emb1.T, emb2.T, emb3.T, level_embed)
    return aug.reshape(L * VEP, DP)


def _idx_body(lv_ref, tk_ref, out_ref):
    out_ref[...] = lv_ref[...] * VEP + tk_ref[...]


def _build_idx(level_ids, token_ids):
    IB = 512
    gidx = pl.pallas_call(
        _idx_body,
        grid=(B // IB,),
        in_specs=[pl.BlockSpec((IB, T), lambda i: (i, 0)) for _ in range(2)],
        out_specs=pl.BlockSpec((IB, T), lambda i: (i, 0)),
        out_shape=jax.ShapeDtypeStruct((B, T), jnp.int32),
    )(level_ids, token_ids)
    return gidx


_MESH = plsc.VectorSubcoreMesh(
    core_axis_name="c", subcore_axis_name="s", num_cores=NC, num_subcores=NS
)


BPW = B // NW
SZ = (104, 96)
T0 = (0, 104)


@functools.partial(
    pl.kernel,
    out_type=jax.ShapeDtypeStruct((B, T, D), jnp.float32),
    mesh=_MESH,
    scratch_types=(
        [pltpu.VMEM((BPW, T), jnp.int32)]
        + [pltpu.VMEM((SZ[0], DP), jnp.float32) for _ in range(NBUF)]
        + [pltpu.SemaphoreType.DMA for _ in range(4)]
    ),
    compiler_params=pltpu.CompilerParams(use_tc_tiling_on_sc=False),
)
def _sc_gather(idx_hbm, table_hbm, out_hbm, idx_v,
               b0, b1, b2, b3, gs0, gs1, ws0, ws1):
    bufs = ((b0, b1), (b2, b3))
    gsem = (gs0, gs1)
    wsem = (ws0, ws1)
    wid = lax.axis_index("s") * NC + lax.axis_index("c")
    row0 = wid * BPW

    pltpu.sync_copy(idx_hbm.at[pl.ds(row0, BPW)], idx_v)

    def chunk_idx(r, b):
        return idx_v.at[r, pl.ds(T0[b], SZ[b])]

    def chunk_out(r, b):
        return out_hbm.at[row0 + r, pl.ds(T0[b], SZ[b]), :]

    def buf_g(g, b):
        return bufs[g][b].at[pl.ds(0, SZ[b]), :]

    def buf_w(g, b):
        return bufs[g][b].at[pl.ds(0, SZ[b]), pl.ds(0, D)]

    for g in range(2):
        for b in range(GRP):
            pltpu.async_copy(table_hbm.at[chunk_idx(g, b)], buf_g(g, b), gsem[g])

    def do_round(r, g):
        for b in range(GRP):
            pltpu.make_async_copy(
                table_hbm.at[chunk_idx(0, b)], buf_g(g, b), gsem[g]
            ).wait()
        for b in range(GRP):
            pltpu.async_copy(buf_w(g, b), chunk_out(r, b), wsem[g])
        for b in range(GRP):
            pltpu.make_async_copy(buf_w(g, b), chunk_out(r, b), wsem[g]).wait()

        @pl.when(r + 2 < BPW)
        def _():
            for b in range(GRP):
                pltpu.async_copy(
                    table_hbm.at[chunk_idx(r + 2, b)], buf_g(g, b), gsem[g]
                )

    def outer(i, carry):
        do_round(2 * i, 0)
        do_round(2 * i + 1, 1)
        return carry

    lax.fori_loop(0, BPW // 2, outer, None)


def kernel(level_ids, token_ids, emb0, emb1, emb2, emb3, level_embed):
    level_ids = level_ids.astype(jnp.int32)
    token_ids = token_ids.astype(jnp.int32)
    table = _build_table(emb0, emb1, emb2, emb3, level_embed)
    gidx = _build_idx(level_ids, token_ids)
    return _sc_gather(gidx, table)


# speedup vs baseline: 1.0608x; 1.0608x over previous
import functools

import jax
import jax.numpy as jnp
from jax import lax
from jax.experimental import pallas as pl
from jax.experimental.pallas import tpu as pltpu
from jax.experimental.pallas import tpu_sc as plsc

B, T, D, L, V = 4096, 200, 64, 4, 100000
VE = V + 2
N = B * T
NC, NS = 2, 16
NW = NC * NS
PER_W = N // NW
CH = 128
NCH = PER_W // CH
GRP = 2
NGROUPS = 3
NBUF = NGROUPS * GRP

BC = 1024
VEP = 100352
DP = 2 * D


def _aug_body(e0_ref, e1_ref, e2_ref, e3_ref, lv_ref, out_ref):
    lv = lv_ref[...]
    for l, e_ref in enumerate((e0_ref, e1_ref, e2_ref, e3_ref)):
        row = e_ref[...].T + lv[l, :]
        out_ref[l, :, :] = jnp.concatenate([row, row], axis=-1)


def _build_table(emb0, emb1, emb2, emb3, level_embed):
    grid = (VEP // BC,)
    aug = pl.pallas_call(
        _aug_body,
        grid=grid,
        in_specs=[pl.BlockSpec((D, BC), lambda i: (0, i)) for _ in range(4)]
        + [pl.BlockSpec((L, D), lambda i: (0, 0))],
        out_specs=pl.BlockSpec((L, BC, DP), lambda i: (0, i, 0)),
        out_shape=jax.ShapeDtypeStruct((L, VEP, DP), jnp.float32),
    )(emb0.T, emb1.T, emb2.T, emb3.T, level_embed)
    return aug.reshape(L * VEP, DP)


def _idx_body(lv_ref, tk_ref, out_ref):
    out_ref[...] = lv_ref[...] * VEP + tk_ref[...]


def _build_idx(level_ids, token_ids):
    IB = 512
    gidx = pl.pallas_call(
        _idx_body,
        grid=(B // IB,),
        in_specs=[pl.BlockSpec((IB, T), lambda i: (i, 0)) for _ in range(2)],
        out_specs=pl.BlockSpec((IB, T), lambda i: (i, 0)),
        out_shape=jax.ShapeDtypeStruct((B, T), jnp.int32),
    )(level_ids, token_ids)
    return gidx


_MESH = plsc.VectorSubcoreMesh(
    core_axis_name="c", subcore_axis_name="s", num_cores=NC, num_subcores=NS
)


NSPLIT = 1
BH = B // NSPLIT
BPW = BH // NW
SZ = (104, 96)
T0 = (0, 104)


def _sc_gather_body(half, idx_hbm, table_hbm, out_hbm, idx_v,
                    b0, b1, b2, b3, b4, b5, gs0, gs1, gs2, ws0, ws1, ws2):
    bufs = ((b0, b1), (b2, b3), (b4, b5))
    gsem = (gs0, gs1, gs2)
    wsem = (ws0, ws1, ws2)
    wid = lax.axis_index("s") * NC + lax.axis_index("c")
    src0 = half * BH + wid * BPW
    row0 = wid * BPW

    pltpu.sync_copy(idx_hbm.at[pl.ds(src0, BPW)], idx_v)

    def chunk_idx(r, b):
        return idx_v.at[r, pl.ds(T0[b], SZ[b])]

    def chunk_out(r, b):
        return out_hbm.at[row0 + r, pl.ds(T0[b], SZ[b]), :]

    def buf_g(g, b):
        return bufs[g][b].at[pl.ds(0, SZ[b]), :]

    def buf_w(g, b):
        return bufs[g][b].at[pl.ds(0, SZ[b]), pl.ds(0, D)]

    for g in range(NGROUPS):
        for b in range(GRP):
            pltpu.async_copy(table_hbm.at[chunk_idx(g, b)], buf_g(g, b), gsem[g])

    def do_round(r, g):
        for b in range(GRP):
            pltpu.make_async_copy(
                table_hbm.at[chunk_idx(0, b)], buf_g(g, b), gsem[g]
            ).wait()
        for b in range(GRP):
            pltpu.async_copy(buf_w(g, b), chunk_out(r, b), wsem[g])
        for b in range(GRP):
            pltpu.make_async_copy(buf_w(g, b), chunk_out(r, b), wsem[g]).wait()

        @pl.when(r + NGROUPS < BPW)
        def _():
            for b in range(GRP):
                pltpu.async_copy(
                    table_hbm.at[chunk_idx(r + NGROUPS, b)], buf_g(g, b), gsem[g]
                )

    def outer(i, carry):
        for g in range(NGROUPS):
            do_round(NGROUPS * i + g, g)
        return carry

    nfull = BPW // NGROUPS
    lax.fori_loop(0, nfull, outer, None)
    for r in range(nfull * NGROUPS, BPW):
        do_round(r, r % NGROUPS)


_SC_SCRATCH = (
    [pltpu.VMEM((BPW, T), jnp.int32)]
    + [pltpu.VMEM((SZ[0], DP), jnp.float32) for _ in range(NBUF)]
    + [pltpu.SemaphoreType.DMA for _ in range(2 * NGROUPS)]
)

_sc_gather_halves = [
    pl.kernel(
        functools.partial(_sc_gather_body, h),
        out_type=jax.ShapeDtypeStruct((BH, T, D), jnp.float32),
        mesh=_MESH,
        scratch_types=_SC_SCRATCH,
        compiler_params=pltpu.CompilerParams(use_tc_tiling_on_sc=False),
        name=f"sc_gather_h{h}",
    )
    for h in range(NSPLIT)
]


def kernel(level_ids, token_ids, emb0, emb1, emb2, emb3, level_embed):
    level_ids = level_ids.astype(jnp.int32)
    token_ids = token_ids.astype(jnp.int32)
    table = _build_table(emb0, emb1, emb2, emb3, level_embed)
    gidx = _build_idx(level_ids, token_ids)
    halves = [g(gidx, table) for g in _sc_gather_halves]
    if NSPLIT == 1:
        return halves[0]
    out = jnp.zeros((B, T, D), jnp.float32)
    for h, part in enumerate(halves):
        out = lax.dynamic_update_slice(out, part, (h * BH, 0, 0))
    return out


# speedup vs baseline: 1.0962x; 1.0334x over previous
import functools

import jax
import jax.numpy as jnp
from jax import lax
from jax.experimental import pallas as pl
from jax.experimental.pallas import tpu as pltpu
from jax.experimental.pallas import tpu_sc as plsc

B, T, D, L, V = 4096, 200, 64, 4, 100000
VE = V + 2
N = B * T
NC, NS = 2, 16
NW = NC * NS
PER_W = N // NW
CH = 128
NCH = PER_W // CH
GRP = 2
NGROUPS = 3
NBUF = NGROUPS * GRP

BC = 2048
VEP = 100352
DP = 2 * D


def _aug_body(e0_ref, e1_ref, e2_ref, e3_ref, lv_ref, out_ref):
    lv = lv_ref[...]
    for l, e_ref in enumerate((e0_ref, e1_ref, e2_ref, e3_ref)):
        row = e_ref[...].T + lv[l, :]
        out_ref[l, :, :] = jnp.concatenate([row, row], axis=-1)


def _build_table(emb0, emb1, emb2, emb3, level_embed):
    grid = (VEP // BC,)
    aug = pl.pallas_call(
        _aug_body,
        grid=grid,
        in_specs=[pl.BlockSpec((D, BC), lambda i: (0, i)) for _ in range(4)]
        + [pl.BlockSpec((L, D), lambda i: (0, 0))],
        out_specs=pl.BlockSpec((L, BC, DP), lambda i: (0, i, 0)),
        out_shape=jax.ShapeDtypeStruct((L, VEP, DP), jnp.float32),
    )(emb0.T, emb1.T, emb2.T, emb3.T, level_embed)
    return aug.reshape(L * VEP, DP)


def _idx_body(lv_ref, tk_ref, out_ref):
    out_ref[...] = lv_ref[...] * VEP + tk_ref[...]


def _build_idx(level_ids, token_ids):
    IB = 512
    gidx = pl.pallas_call(
        _idx_body,
        grid=(B // IB,),
        in_specs=[pl.BlockSpec((IB, T), lambda i: (i, 0)) for _ in range(2)],
        out_specs=pl.BlockSpec((IB, T), lambda i: (i, 0)),
        out_shape=jax.ShapeDtypeStruct((B, T), jnp.int32),
    )(level_ids, token_ids)
    return gidx


_MESH = plsc.VectorSubcoreMesh(
    core_axis_name="c", subcore_axis_name="s", num_cores=NC, num_subcores=NS
)


NSPLIT = 1
BH = B // NSPLIT
BPW = BH // NW
SZ = (128, 72)
T0 = (0, 128)


def _sc_gather_body(half, idx_hbm, table_hbm, out_hbm, idx_v,
                    b0, b1, b2, b3, b4, b5, gs0, gs1, gs2, ws0, ws1, ws2):
    bufs = ((b0, b1), (b2, b3), (b4, b5))
    gsem = (gs0, gs1, gs2)
    wsem = (ws0, ws1, ws2)
    wid = lax.axis_index("s") * NC + lax.axis_index("c")
    src0 = half * BH + wid * BPW
    row0 = wid * BPW

    pltpu.sync_copy(idx_hbm.at[pl.ds(src0, BPW)], idx_v)

    def chunk_idx(r, b):
        return idx_v.at[r, pl.ds(T0[b], SZ[b])]

    def chunk_out(r, b):
        return out_hbm.at[row0 + r, pl.ds(T0[b], SZ[b]), :]

    def buf_g(g, b):
        return bufs[g][b].at[pl.ds(0, SZ[b]), :]

    def buf_w(g, b):
        return bufs[g][b].at[pl.ds(0, SZ[b]), pl.ds(0, D)]

    for g in range(NGROUPS):
        for b in range(GRP):
            pltpu.async_copy(table_hbm.at[chunk_idx(g, b)], buf_g(g, b), gsem[g])

    def do_round(r, g):
        for b in range(GRP):
            pltpu.make_async_copy(
                table_hbm.at[chunk_idx(0, b)], buf_g(g, b), gsem[g]
            ).wait()
        for b in range(GRP):
            pltpu.async_copy(buf_w(g, b), chunk_out(r, b), wsem[g])
        for b in range(GRP):
            pltpu.make_async_copy(buf_w(g, b), chunk_out(r, b), wsem[g]).wait()

        @pl.when(r + NGROUPS < BPW)
        def _():
            for b in range(GRP):
                pltpu.async_copy(
                    table_hbm.at[chunk_idx(r + NGROUPS, b)], buf_g(g, b), gsem[g]
                )

    def outer(i, carry):
        for g in range(NGROUPS):
            do_round(NGROUPS * i + g, g)
        return carry

    nfull = BPW // NGROUPS
    lax.fori_loop(0, nfull, outer, None)
    for r in range(nfull * NGROUPS, BPW):
        do_round(r, r % NGROUPS)


_SC_SCRATCH = (
    [pltpu.VMEM((BPW, T), jnp.int32)]
    + [pltpu.VMEM((SZ[0], DP), jnp.float32) for _ in range(NBUF)]
    + [pltpu.SemaphoreType.DMA for _ in range(2 * NGROUPS)]
)

_sc_gather_halves = [
    pl.kernel(
        functools.partial(_sc_gather_body, h),
        out_type=jax.ShapeDtypeStruct((BH, T, D), jnp.float32),
        mesh=_MESH,
        scratch_types=_SC_SCRATCH,
        compiler_params=pltpu.CompilerParams(use_tc_tiling_on_sc=False),
        name=f"sc_gather_h{h}",
    )
    for h in range(NSPLIT)
]


def kernel(level_ids, token_ids, emb0, emb1, emb2, emb3, level_embed):
    level_ids = level_ids.astype(jnp.int32)
    token_ids = token_ids.astype(jnp.int32)
    table = _build_table(emb0, emb1, emb2, emb3, level_embed)
    gidx = _build_idx(level_ids, token_ids)
    halves = [g(gidx, table) for g in _sc_gather_halves]
    if NSPLIT == 1:
        return halves[0]
    out = jnp.zeros((B, T, D), jnp.float32)
    for h, part in enumerate(halves):
        out = lax.dynamic_update_slice(out, part, (h * BH, 0, 0))
    return out


# speedup vs baseline: 1.1101x; 1.0126x over previous
import functools

import jax
import jax.numpy as jnp
from jax import lax
from jax.experimental import pallas as pl
from jax.experimental.pallas import tpu as pltpu
from jax.experimental.pallas import tpu_sc as plsc

B, T, D, L, V = 4096, 200, 64, 4, 100000
VE = V + 2
N = B * T
NC, NS = 2, 16
NW = NC * NS
PER_W = N // NW
CH = 128
NCH = PER_W // CH
GRP = 2
NGROUPS = 3
NBUF = NGROUPS * GRP

BC = 2048
VEP = 100352
DP = 2 * D


def _aug_body(e0_ref, e1_ref, e2_ref, e3_ref, lv_ref, out_ref):
    lv = lv_ref[...]
    for l, e_ref in enumerate((e0_ref, e1_ref, e2_ref, e3_ref)):
        row = e_ref[...].T + lv[l, :]
        out_ref[l, :, :] = jnp.concatenate([row, row], axis=-1)


def _build_table(emb0, emb1, emb2, emb3, level_embed):
    grid = (VEP // BC,)
    aug = pl.pallas_call(
        _aug_body,
        grid=grid,
        in_specs=[pl.BlockSpec((D, BC), lambda i: (0, i)) for _ in range(4)]
        + [pl.BlockSpec((L, D), lambda i: (0, 0))],
        out_specs=pl.BlockSpec((L, BC, DP), lambda i: (0, i, 0)),
        out_shape=jax.ShapeDtypeStruct((L, VEP, DP), jnp.float32),
    )(emb0.T, emb1.T, emb2.T, emb3.T, level_embed)
    return aug.reshape(L * VEP, DP)


def _idx_body(lv_ref, tk_ref, out_ref):
    out_ref[...] = lv_ref[...].T * VEP + tk_ref[...].T


def _build_idx(level_ids, token_ids):
    IB = 512
    gidx = pl.pallas_call(
        _idx_body,
        grid=(B // IB,),
        in_specs=[pl.BlockSpec((T, IB), lambda i: (0, i)) for _ in range(2)],
        out_specs=pl.BlockSpec((IB, T), lambda i: (i, 0)),
        out_shape=jax.ShapeDtypeStruct((B, T), jnp.int32),
    )(level_ids.T, token_ids.T)
    return gidx


_MESH = plsc.VectorSubcoreMesh(
    core_axis_name="c", subcore_axis_name="s", num_cores=NC, num_subcores=NS
)


NSPLIT = 1
BH = B // NSPLIT
BPW = BH // NW
SZ = (128, 72)
T0 = (0, 128)


def _sc_gather_body(half, idx_hbm, table_hbm, out_hbm, idx_v,
                    b0, b1, b2, b3, b4, b5, gs0, gs1, gs2, ws0, ws1, ws2):
    bufs = ((b0, b1), (b2, b3), (b4, b5))
    gsem = (gs0, gs1, gs2)
    wsem = (ws0, ws1, ws2)
    wid = lax.axis_index("s") * NC + lax.axis_index("c")
    src0 = half * BH + wid * BPW
    row0 = wid * BPW

    pltpu.sync_copy(idx_hbm.at[pl.ds(src0, BPW)], idx_v)

    def chunk_idx(r, b):
        return idx_v.at[r, pl.ds(T0[b], SZ[b])]

    def chunk_out(r, b):
        return out_hbm.at[row0 + r, pl.ds(T0[b], SZ[b]), :]

    def buf_g(g, b):
        return bufs[g][b].at[pl.ds(0, SZ[b]), :]

    def buf_w(g, b):
        return bufs[g][b].at[pl.ds(0, SZ[b]), pl.ds(0, D)]

    for g in range(NGROUPS):
        for b in range(GRP):
            pltpu.async_copy(table_hbm.at[chunk_idx(g, b)], buf_g(g, b), gsem[g])

    def do_round(r, g):
        for b in range(GRP):
            pltpu.make_async_copy(
                table_hbm.at[chunk_idx(0, b)], buf_g(g, b), gsem[g]
            ).wait()
        for b in range(GRP):
            pltpu.async_copy(buf_w(g, b), chunk_out(r, b), wsem[g])
        for b in range(GRP):
            pltpu.make_async_copy(buf_w(g, b), chunk_out(r, b), wsem[g]).wait()

        @pl.when(r + NGROUPS < BPW)
        def _():
            for b in range(GRP):
                pltpu.async_copy(
                    table_hbm.at[chunk_idx(r + NGROUPS, b)], buf_g(g, b), gsem[g]
                )

    def outer(i, carry):
        for g in range(NGROUPS):
            do_round(NGROUPS * i + g, g)
        return carry

    nfull = BPW // NGROUPS
    lax.fori_loop(0, nfull, outer, None)
    for r in range(nfull * NGROUPS, BPW):
        do_round(r, r % NGROUPS)


_SC_SCRATCH = (
    [pltpu.VMEM((BPW, T), jnp.int32)]
    + [pltpu.VMEM((SZ[0], DP), jnp.float32) for _ in range(NBUF)]
    + [pltpu.SemaphoreType.DMA for _ in range(2 * NGROUPS)]
)

_sc_gather_halves = [
    pl.kernel(
        functools.partial(_sc_gather_body, h),
        out_type=jax.ShapeDtypeStruct((BH, T, D), jnp.float32),
        mesh=_MESH,
        scratch_types=_SC_SCRATCH,
        compiler_params=pltpu.CompilerParams(use_tc_tiling_on_sc=False),
        name=f"sc_gather_h{h}",
    )
    for h in range(NSPLIT)
]


def kernel(level_ids, token_ids, emb0, emb1, emb2, emb3, level_embed):
    level_ids = level_ids.astype(jnp.int32)
    token_ids = token_ids.astype(jnp.int32)
    table = _build_table(emb0, emb1, emb2, emb3, level_embed)
    gidx = _build_idx(level_ids, token_ids)
    halves = [g(gidx, table) for g in _sc_gather_halves]
    if NSPLIT == 1:
        return halves[0]
    out = jnp.zeros((B, T, D), jnp.float32)
    for h, part in enumerate(halves):
        out = lax.dynamic_update_slice(out, part, (h * BH, 0, 0))
    return out


# speedup vs baseline: 1.1110x; 1.0008x over previous
import functools

import jax
import jax.numpy as jnp
from jax import lax
from jax.experimental import pallas as pl
from jax.experimental.pallas import tpu as pltpu
from jax.experimental.pallas import tpu_sc as plsc

B, T, D, L, V = 4096, 200, 64, 4, 100000
VE = V + 2
N = B * T
NC, NS = 2, 16
NW = NC * NS
GRP = 2
NGROUPS = 3
NBUF = NGROUPS * GRP

BC = 2048
VEP = 100352
DP = 2 * D

BPW = B // NW
SZ = (128, 72)
T0 = (0, 128)


def _aug_body(e0_ref, e1_ref, e2_ref, e3_ref, lv_ref, out_ref):
    lv = lv_ref[...]
    for l, e_ref in enumerate((e0_ref, e1_ref, e2_ref, e3_ref)):
        row = e_ref[...].T + lv[l, :]
        out_ref[l, :, :] = jnp.concatenate([row, row], axis=-1)


def _build_table(emb0, emb1, emb2, emb3, level_embed):
    aug = pl.pallas_call(
        _aug_body,
        grid=(VEP // BC,),
        in_specs=[pl.BlockSpec((D, BC), lambda i: (0, i)) for _ in range(4)]
        + [pl.BlockSpec((L, D), lambda i: (0, 0))],
        out_specs=pl.BlockSpec((L, BC, DP), lambda i: (0, i, 0)),
        out_shape=jax.ShapeDtypeStruct((L, VEP, DP), jnp.float32),
    )(emb0.T, emb1.T, emb2.T, emb3.T, level_embed)
    return aug.reshape(L * VEP, DP)


def _idx_body(lv_ref, tk_ref, out_ref):
    out_ref[...] = lv_ref[...].T * VEP + tk_ref[...].T


def _build_idx(level_ids, token_ids):
    IB = 512
    return pl.pallas_call(
        _idx_body,
        grid=(B // IB,),
        in_specs=[pl.BlockSpec((T, IB), lambda i: (0, i)) for _ in range(2)],
        out_specs=pl.BlockSpec((IB, T), lambda i: (i, 0)),
        out_shape=jax.ShapeDtypeStruct((B, T), jnp.int32),
    )(level_ids.T, token_ids.T)


_MESH = plsc.VectorSubcoreMesh(
    core_axis_name="c", subcore_axis_name="s", num_cores=NC, num_subcores=NS
)


@functools.partial(
    pl.kernel,
    out_type=jax.ShapeDtypeStruct((B, T, D), jnp.float32),
    mesh=_MESH,
    scratch_types=(
        [pltpu.VMEM((BPW, T), jnp.int32)]
        + [pltpu.VMEM((SZ[0], DP), jnp.float32) for _ in range(NBUF)]
        + [pltpu.SemaphoreType.DMA for _ in range(2 * NGROUPS)]
    ),
    compiler_params=pltpu.CompilerParams(use_tc_tiling_on_sc=False),
    name="sc_gather",
)
def _sc_gather(idx_hbm, table_hbm, out_hbm, idx_v,
               b0, b1, b2, b3, b4, b5, gs0, gs1, gs2, ws0, ws1, ws2):
    bufs = ((b0, b1), (b2, b3), (b4, b5))
    gsem = (gs0, gs1, gs2)
    wsem = (ws0, ws1, ws2)
    wid = lax.axis_index("s") * NC + lax.axis_index("c")
    row0 = wid * BPW

    pltpu.sync_copy(idx_hbm.at[pl.ds(row0, BPW)], idx_v)

    def chunk_idx(r, b):
        return idx_v.at[r, pl.ds(T0[b], SZ[b])]

    def chunk_out(r, b):
        return out_hbm.at[row0 + r, pl.ds(T0[b], SZ[b]), :]

    def buf_g(g, b):
        return bufs[g][b].at[pl.ds(0, SZ[b]), :]

    def buf_w(g, b):
        return bufs[g][b].at[pl.ds(0, SZ[b]), pl.ds(0, D)]

    for g in range(NGROUPS):
        for b in range(GRP):
            pltpu.async_copy(table_hbm.at[chunk_idx(g, b)], buf_g(g, b), gsem[g])

    def do_round(r, g):
        for b in range(GRP):
            pltpu.make_async_copy(
                table_hbm.at[chunk_idx(0, b)], buf_g(g, b), gsem[g]
            ).wait()
        for b in range(GRP):
            pltpu.async_copy(buf_w(g, b), chunk_out(r, b), wsem[g])
        for b in range(GRP):
            pltpu.make_async_copy(buf_w(g, b), chunk_out(r, b), wsem[g]).wait()

        @pl.when(r + NGROUPS < BPW)
        def _():
            for b in range(GRP):
                pltpu.async_copy(
                    table_hbm.at[chunk_idx(r + NGROUPS, b)], buf_g(g, b), gsem[g]
                )

    def outer(i, carry):
        for g in range(NGROUPS):
            do_round(NGROUPS * i + g, g)
        return carry

    nfull = BPW // NGROUPS
    lax.fori_loop(0, nfull, outer, None)
    for r in range(nfull * NGROUPS, BPW):
        do_round(r, r % NGROUPS)


def kernel(level_ids, token_ids, emb0, emb1, emb2, emb3, level_embed):
    level_ids = level_ids.astype(jnp.int32)
    token_ids = token_ids.astype(jnp.int32)
    table = _build_table(emb0, emb1, emb2, emb3, level_embed)
    gidx = _build_idx(level_ids, token_ids)
    return _sc_gather(gidx, table)
